# baseline (speedup 1.0000x reference)
import functools

import jax
import jax.numpy as jnp
from jax import lax
from jax.experimental import pallas as pl
from jax.experimental.pallas import tpu as pltpu
from jax.experimental.pallas import tpu_sc as plsc

N = 10000
E = 320000
D = 128
OUT = 16

NC = 2
NS = 16
NW = NC * NS
EPT = E // NW
K = 80
CH = EPT // K
NBUF = 4
PD = NBUF // 2
RPT = N // NS


def _mm_body(x_ref, w_ref, o_ref):
    o_ref[...] = jnp.dot(x_ref[...], w_ref[...], preferred_element_type=jnp.float32)


CRPW = N // NW
CTAIL = N - NW * CRPW


def _combine_body(p_hbm, out_hbm, a_v, b_v, at_v, bt_v):
    cid = lax.axis_index("c")
    sid = lax.axis_index("s")
    wid = sid * NC + cid
    base = wid * CRPW
    pltpu.sync_copy(p_hbm.at[0, pl.ds(base, CRPW)], a_v)
    pltpu.sync_copy(p_hbm.at[1, pl.ds(base, CRPW)], b_v)

    def body(i, c):
        a_v[i] = jnp.maximum(a_v[i] + b_v[i], 0.0)
        return c

    lax.fori_loop(0, CRPW, body, 0, unroll=8)
    pltpu.sync_copy(a_v, out_hbm.at[pl.ds(base, CRPW)])

    @pl.when(wid == NW - 1)
    def _():
        tb = NW * CRPW
        pltpu.sync_copy(p_hbm.at[0, pl.ds(tb, CTAIL)], at_v)
        pltpu.sync_copy(p_hbm.at[1, pl.ds(tb, CTAIL)], bt_v)
        for i in range(CTAIL):
            at_v[i] = jnp.maximum(at_v[i] + bt_v[i], 0.0)
        pltpu.sync_copy(at_v, out_hbm.at[pl.ds(tb, CTAIL)])


_combine_kernel = functools.partial(
    pl.kernel,
    out_type=jax.ShapeDtypeStruct((N, OUT), jnp.float32),
    mesh=plsc.VectorSubcoreMesh(core_axis_name="c", subcore_axis_name="s"),
    compiler_params=pltpu.CompilerParams(use_tc_tiling_on_sc=False),
    scratch_types=[
        pltpu.VMEM((CRPW, OUT), jnp.float32),
        pltpu.VMEM((CRPW, OUT), jnp.float32),
        pltpu.VMEM((CTAIL, OUT), jnp.float32),
        pltpu.VMEM((CTAIL, OUT), jnp.float32),
    ],
)(_combine_body)


def _edge_body(ei_hbm, w_hbm, xw_hbm, out_hbm,
               src_v, dst_v, w_v, rows_v, stage_v, acc_sh, xw_sh, gsem, ssem):
    cid = lax.axis_index("c")
    sid = lax.axis_index("s")
    wid = sid * NC + cid
    ebase = wid * EPT

    zero16 = jnp.zeros((16,), jnp.float32)

    def zinit(i, c):
        stage_v[i] = zero16
        return c

    lax.fori_loop(0, RPT, zinit, 0, unroll=8)
    pltpu.sync_copy(stage_v, acc_sh.at[pl.ds(sid * RPT, RPT)])

    pltpu.sync_copy(ei_hbm.at[0, pl.ds(ebase, EPT)], src_v)
    pltpu.sync_copy(ei_hbm.at[1, pl.ds(ebase, EPT)], dst_v)
    pltpu.sync_copy(w_hbm.at[pl.ds(ebase, EPT)], w_v)
    pltpu.sync_copy(xw_hbm.at[pl.ds(sid * RPT, RPT)],
                    xw_sh.at[pl.ds(sid * RPT, RPT)])
    plsc.subcore_barrier()

    def gather_issue(j, b):
        pltpu.async_copy(xw_sh.at[src_v.at[pl.ds(j * K, K)]], rows_v.at[b],
                         gsem)

    def gather_wait(b):
        pltpu.make_async_copy(xw_hbm.at[pl.ds(0, K)], rows_v.at[b], gsem).wait()

    def scatter_issue(j, b):
        pltpu.async_copy(rows_v.at[b], acc_sh.at[dst_v.at[pl.ds(j * K, K)]],
                         ssem, add=True)

    def scatter_wait(b):
        pltpu.make_async_copy(xw_hbm.at[pl.ds(0, K)], rows_v.at[b], ssem).wait()

    def scale(j, b):
        base = j * K
        for g in range(K // 16):
            wv = w_v[pl.ds(base + g * 16, 16)]
            for l in range(16):
                e = g * 16 + l
                wsp = jnp.take_along_axis(wv, jnp.full((16,), l, jnp.int32),
                                          axis=0)
                rows_v[b, e] = rows_v[b, e] * wsp

    for j0 in range(PD):
        gather_issue(j0, j0)

    def step(j, b, traced):
        if traced:
            @pl.when(j >= PD)
            def _():
                scatter_wait((b + PD) % NBUF)

            @pl.when(j + PD < CH)
            def _():
                gather_issue(j + PD, (b + PD) % NBUF)
        else:
            if j >= PD:
                scatter_wait((b + PD) % NBUF)
            if j + PD < CH:
                gather_issue(j + PD, (b + PD) % NBUF)
        gather_wait(b)
        scale(j, b)
        scatter_issue(j, b)

    def pipe_body(i, c):
        jj = i * NBUF
        for b in range(NBUF):
            step(jj + b, b, True)
        return c

    nfull = (CH // NBUF) * NBUF
    lax.fori_loop(0, CH // NBUF, pipe_body, 0)
    for j in range(nfull, CH):
        step(j, j % NBUF, False)
    for j in range(CH - PD, CH):
        scatter_wait(j % NBUF)
    plsc.subcore_barrier()

    pltpu.sync_copy(acc_sh.at[pl.ds(sid * RPT, RPT)],
                    out_hbm.at[cid, pl.ds(sid * RPT, RPT)])


_edge_kernel = functools.partial(
    pl.kernel,
    out_type=jax.ShapeDtypeStruct((NC, N, OUT), jnp.float32),
    mesh=plsc.VectorSubcoreMesh(core_axis_name="c", subcore_axis_name="s"),
    compiler_params=pltpu.CompilerParams(use_tc_tiling_on_sc=False),
    scratch_types=[
        pltpu.VMEM((EPT,), jnp.int32),
        pltpu.VMEM((EPT,), jnp.int32),
        pltpu.VMEM((EPT,), jnp.float32),
        pltpu.VMEM((NBUF, K, OUT), jnp.float32),
        pltpu.VMEM((RPT, OUT), jnp.float32),
        pltpu.VMEM_SHARED((N, OUT), jnp.float32),
        pltpu.VMEM_SHARED((N, OUT), jnp.float32),
        pltpu.SemaphoreType.DMA,
        pltpu.SemaphoreType.DMA,
    ],
)(_edge_body)


def kernel(inputs, edge_index, edge_weight, kernel):
    xw = pl.pallas_call(
        _mm_body,
        grid=(5,),
        in_specs=[pl.BlockSpec((N // 5, D), lambda i: (i, 0)),
                  pl.BlockSpec((D, OUT), lambda i: (0, 0))],
        out_specs=pl.BlockSpec((N // 5, OUT), lambda i: (i, 0)),
        out_shape=jax.ShapeDtypeStruct((N, OUT), jnp.float32),
    )(inputs, kernel)

    partials = _edge_kernel(edge_index, edge_weight, xw)

    return _combine_kernel(partials)


# speedup vs baseline: 1.0343x; 1.0343x over previous
import functools

import jax
import jax.numpy as jnp
from jax import lax
from jax.experimental import pallas as pl
from jax.experimental.pallas import tpu as pltpu
from jax.experimental.pallas import tpu_sc as plsc

N = 10000
E = 320000
D = 128
OUT = 16

NC = 2
NS = 16
NW = NC * NS
EPT = E // NW
K = 80
CH = EPT // K
NBUF = 4
PD = NBUF // 2
RPT = N // NS


def _mm_body(x_ref, w_ref, o_ref):
    o_ref[...] = jnp.dot(x_ref[...], w_ref[...], preferred_element_type=jnp.float32)


CRPW = N // NW
CTAIL = N - NW * CRPW


def _combine_body(p_hbm, out_hbm, a_v, b_v, at_v, bt_v):
    cid = lax.axis_index("c")
    sid = lax.axis_index("s")
    wid = sid * NC + cid
    base = wid * CRPW
    pltpu.sync_copy(p_hbm.at[0, pl.ds(base, CRPW)], a_v)
    pltpu.sync_copy(p_hbm.at[1, pl.ds(base, CRPW)], b_v)

    def body(i, c):
        a_v[i] = jnp.maximum(a_v[i] + b_v[i], 0.0)
        return c

    lax.fori_loop(0, CRPW, body, 0, unroll=8)
    pltpu.sync_copy(a_v, out_hbm.at[pl.ds(base, CRPW)])

    @pl.when(wid == NW - 1)
    def _():
        tb = NW * CRPW
        pltpu.sync_copy(p_hbm.at[0, pl.ds(tb, CTAIL)], at_v)
        pltpu.sync_copy(p_hbm.at[1, pl.ds(tb, CTAIL)], bt_v)
        for i in range(CTAIL):
            at_v[i] = jnp.maximum(at_v[i] + bt_v[i], 0.0)
        pltpu.sync_copy(at_v, out_hbm.at[pl.ds(tb, CTAIL)])


_combine_kernel = functools.partial(
    pl.kernel,
    out_type=jax.ShapeDtypeStruct((N, OUT), jnp.float32),
    mesh=plsc.VectorSubcoreMesh(core_axis_name="c", subcore_axis_name="s"),
    compiler_params=pltpu.CompilerParams(use_tc_tiling_on_sc=False),
    scratch_types=[
        pltpu.VMEM((CRPW, OUT), jnp.float32),
        pltpu.VMEM((CRPW, OUT), jnp.float32),
        pltpu.VMEM((CTAIL, OUT), jnp.float32),
        pltpu.VMEM((CTAIL, OUT), jnp.float32),
    ],
)(_combine_body)


def _edge_body(ei_hbm, w_hbm, xw_hbm, out_hbm,
               src_v, dst_v, w_v, rows_v, stage_v, acc_sh, xw_sh, gsem, ssem):
    cid = lax.axis_index("c")
    sid = lax.axis_index("s")
    wid = sid * NC + cid
    ebase = wid * EPT

    zero16 = jnp.zeros((16,), jnp.float32)

    def zinit(i, c):
        stage_v[i] = zero16
        return c

    lax.fori_loop(0, RPT, zinit, 0, unroll=8)
    pltpu.sync_copy(stage_v, acc_sh.at[pl.ds(sid * RPT, RPT)])

    pltpu.sync_copy(ei_hbm.at[0, pl.ds(ebase, EPT)], src_v)
    pltpu.sync_copy(ei_hbm.at[1, pl.ds(ebase, EPT)], dst_v)
    pltpu.sync_copy(w_hbm.at[pl.ds(ebase, EPT)], w_v)
    pltpu.sync_copy(xw_hbm.at[pl.ds(sid * RPT, RPT)],
                    xw_sh.at[pl.ds(sid * RPT, RPT)])
    plsc.subcore_barrier()

    def gather_issue(j, b):
        pltpu.async_copy(xw_sh.at[src_v.at[pl.ds(j * K, K)]], rows_v.at[b],
                         gsem)

    def gather_wait(b):
        pltpu.make_async_copy(xw_hbm.at[pl.ds(0, K)], rows_v.at[b], gsem).wait()

    def scatter_issue(j, b):
        pltpu.async_copy(rows_v.at[b], acc_sh.at[dst_v.at[pl.ds(j * K, K)]],
                         ssem, add=True)

    def scatter_wait(b):
        pltpu.make_async_copy(xw_hbm.at[pl.ds(0, K)], rows_v.at[b], ssem).wait()

    def scale(j, b):
        base = j * K
        for g in range(K // 16):
            wv = w_v[pl.ds(base + g * 16, 16)]
            for l in range(16):
                e = g * 16 + l
                wsp = jnp.take_along_axis(wv, jnp.full((16,), l, jnp.int32),
                                          axis=0)
                rows_v[b, e] = rows_v[b, e] * wsp

    for j0 in range(PD):
        gather_issue(j0, j0)

    def step(j, b, traced):
        if traced:
            @pl.when(j >= PD)
            def _():
                scatter_wait((b + PD) % NBUF)

            @pl.when(j + PD < CH)
            def _():
                gather_issue(j + PD, (b + PD) % NBUF)
        else:
            if j >= PD:
                scatter_wait((b + PD) % NBUF)
            if j + PD < CH:
                gather_issue(j + PD, (b + PD) % NBUF)
        gather_wait(b)
        scale(j, b)
        scatter_issue(j, b)

    def pipe_body(i, c):
        jj = i * NBUF
        for b in range(NBUF):
            step(jj + b, b, True)
        return c

    nfull = (CH // NBUF) * NBUF
    lax.fori_loop(0, CH // NBUF, pipe_body, 0)
    for j in range(nfull, CH):
        step(j, j % NBUF, False)
    for j in range(CH - PD, CH):
        scatter_wait(j % NBUF)
    plsc.subcore_barrier()

    pltpu.sync_copy(acc_sh.at[pl.ds(sid * RPT, RPT)],
                    out_hbm.at[cid, pl.ds(sid * RPT, RPT)])


_edge_kernel = functools.partial(
    pl.kernel,
    out_type=jax.ShapeDtypeStruct((NC, N, OUT), jnp.float32),
    mesh=plsc.VectorSubcoreMesh(core_axis_name="c", subcore_axis_name="s"),
    compiler_params=pltpu.CompilerParams(use_tc_tiling_on_sc=False),
    scratch_types=[
        pltpu.VMEM((EPT,), jnp.int32),
        pltpu.VMEM((EPT,), jnp.int32),
        pltpu.VMEM((EPT,), jnp.float32),
        pltpu.VMEM((NBUF, K, OUT), jnp.float32),
        pltpu.VMEM((RPT, OUT), jnp.float32),
        pltpu.VMEM_SHARED((N, OUT), jnp.float32),
        pltpu.VMEM_SHARED((N, OUT), jnp.float32),
        pltpu.SemaphoreType.DMA,
        pltpu.SemaphoreType.DMA,
    ],
)(_edge_body)


def kernel(inputs, edge_index, edge_weight, kernel):
    xw = pl.pallas_call(
        _mm_body,
        grid=(2,),
        in_specs=[pl.BlockSpec((N // 2, D), lambda i: (i, 0)),
                  pl.BlockSpec((D, OUT), lambda i: (0, 0))],
        out_specs=pl.BlockSpec((N // 2, OUT), lambda i: (i, 0)),
        out_shape=jax.ShapeDtypeStruct((N, OUT), jnp.float32),
    )(inputs, kernel)

    partials = _edge_kernel(edge_index, edge_weight, xw)

    return _combine_kernel(partials)
